# Initial kernel scaffold; baseline (speedup 1.0000x reference)
import jax
import jax.numpy as jnp
from jax.experimental import pallas as pl


def kernel(x, Wr1, br1, Wr2, br2, W1, b1, W2, b2):
    raise NotImplementedError("write your pallas kernel here")


# speedup vs baseline: 1.1186x; 1.1186x over previous
import jax
import jax.numpy as jnp
from jax.experimental import pallas as pl
from jax.experimental.pallas import tpu as pltpu

T, C, E, H = 2048, 768, 8, 2688
HB = 3
HBK = H // HB


def _router_body(x_ref, wr1_ref, br1_ref, wr2_ref, br2_ref, w_ref):
    x = x_ref[...]
    rh = jnp.dot(x, wr1_ref[...], preferred_element_type=jnp.float32)
    rh = jnp.maximum(rh + br1_ref[...], 0.0)
    logits = jnp.dot(rh, wr2_ref[...], preferred_element_type=jnp.float32)
    logits = logits + br2_ref[...]
    m = jnp.max(logits, axis=-1, keepdims=True)
    ex = jnp.exp(logits - m)
    probs = ex / jnp.sum(ex, axis=-1, keepdims=True)
    idx = jax.lax.broadcasted_iota(jnp.int32, probs.shape, 1)
    m1 = jnp.max(probs, axis=-1, keepdims=True)
    i1 = jnp.min(jnp.where(probs == m1, idx, E), axis=-1, keepdims=True)
    sel1 = idx == i1
    pm = jnp.where(sel1, -jnp.inf, probs)
    m2 = jnp.max(pm, axis=-1, keepdims=True)
    i2 = jnp.min(jnp.where(pm == m2, idx, E), axis=-1, keepdims=True)
    sel2 = idx == i2
    mask = sel1 | sel2
    w_ref[...] = jnp.where(mask, probs / (m1 + m2), 0.0)


def _expert_body(w_ref, x_ref, w1_ref, b1_ref, w2_ref, b2_ref, out_ref,
                 acc_ref):
    e = pl.program_id(0)
    hb = pl.program_id(1)
    x = x_ref[...]
    xb = x.astype(jnp.bfloat16)
    w1 = w1_ref[0].astype(jnp.bfloat16)
    h = jnp.dot(xb, w1, preferred_element_type=jnp.float32)
    h = jnp.maximum(h + b1_ref[0], 0.0)
    w2 = w2_ref[0].astype(jnp.bfloat16)
    part = jnp.dot(h.astype(jnp.bfloat16), w2,
                   preferred_element_type=jnp.float32)

    @pl.when(hb == 0)
    def _():
        acc_ref[...] = part

    @pl.when(hb > 0)
    def _():
        acc_ref[...] += part

    @pl.when(hb == HB - 1)
    def _():
        lane = jax.lax.broadcasted_iota(jnp.int32, (T, E), 1)
        wcol = jnp.sum(jnp.where(lane == e, w_ref[...], 0.0), axis=1,
                       keepdims=True)
        contrib = wcol * (acc_ref[...] + b2_ref[0])

        @pl.when(e == 0)
        def _():
            out_ref[...] = x + contrib

        @pl.when(e > 0)
        def _():
            out_ref[...] += contrib


def kernel(x, Wr1, br1, Wr2, br2, W1, b1, W2, b2):
    x2 = x.reshape(T, C)
    w_full = pl.pallas_call(
        _router_body,
        out_shape=jax.ShapeDtypeStruct((T, E), jnp.float32),
    )(x2, Wr1, br1.reshape(1, C // 2), Wr2, br2.reshape(1, E))

    out = pl.pallas_call(
        _expert_body,
        grid=(E, HB),
        in_specs=[
            pl.BlockSpec((T, E), lambda e, hb: (0, 0)),
            pl.BlockSpec((T, C), lambda e, hb: (0, 0)),
            pl.BlockSpec((1, C, HBK), lambda e, hb: (e, 0, hb)),
            pl.BlockSpec((1, 1, HBK), lambda e, hb: (e, 0, hb)),
            pl.BlockSpec((1, HBK, C), lambda e, hb: (e, hb, 0)),
            pl.BlockSpec((1, 1, C), lambda e, hb: (e, 0, 0)),
        ],
        out_specs=pl.BlockSpec((T, C), lambda e, hb: (0, 0)),
        out_shape=jax.ShapeDtypeStruct((T, C), jnp.float32),
        scratch_shapes=[pltpu.VMEM((T, C), jnp.float32)],
        compiler_params=pltpu.CompilerParams(
            dimension_semantics=("arbitrary", "arbitrary"),
        ),
    )(w_full, x2, W1, b1.reshape(E, 1, H), W2, b2.reshape(E, 1, C))
    return out.reshape(1, T, C)
